# baseline (speedup 1.0000x reference)
import jax
import jax.numpy as jnp
from jax import lax
from jax.experimental import pallas as pl
from jax.experimental.pallas import tpu as pltpu
from jax.experimental.pallas import tpu_sc as plsc

_N = 10000
_D = 128
_E = 320000
_OUT = 40

_NC = 2
_NS = 16
_NW = _NC * _NS
_CHUNK = 64
_NBUF = 5
_NIDX = 10
_LG = 3
_LS = 2
_LI = 5
_GRP = _NIDX
_NCH = 160
_NCALLS = 2
_EPAD = _NS * _NCH * _NCALLS * _CHUNK
_NPAD = ((_N + _NS * 8 - 1) // (_NS * 8)) * (_NS * 8)
_RPT = _NPAD // _NS


def _sc_segsum_body(x_hbm, src_hbm, dst_hbm, init_hbm, out_hbm,
                    sidx, didx, rows, acc, gsems, ssems, isems):
    c = lax.axis_index("c")
    s = lax.axis_index("s")

    def pipeline(nchunk, base):
        def idx_start(i, q):
            pltpu.async_copy(src_hbm.at[base + i], sidx.at[q], isems[q])
            pltpu.async_copy(dst_hbm.at[base + i], didx.at[q], isems[q])

        def idx_wait(i, q):
            pltpu.make_async_copy(src_hbm.at[base + i], sidx.at[q],
                                  isems[q]).wait()
            pltpu.make_async_copy(dst_hbm.at[base + i], didx.at[q],
                                  isems[q]).wait()

        def gather_start(i, q, r):
            pltpu.async_copy(x_hbm.at[sidx.at[q]], rows.at[r], gsems[r])

        def gather_wait(i, q, r):
            pltpu.make_async_copy(x_hbm.at[sidx.at[q]], rows.at[r],
                                  gsems[r]).wait()

        def scatter_start(i, q, r):
            pltpu.async_copy(rows.at[r], acc.at[didx.at[q]], ssems[r],
                             add=True)

        def scatter_wait(i, q, r):
            pltpu.make_async_copy(rows.at[r], acc.at[didx.at[q]],
                                  ssems[r]).wait()

        def emit_chunk(i, u, first, last):
            if (not first) or u >= _LS:
                scatter_wait(i - _LS, (u - _LS) % _NIDX, (u - _LS) % _NBUF)
            if (not last) or u < _GRP - _LI:
                idx_start(i + _LI, (u + _LI) % _NIDX)
            if (not last) or u < _GRP - _LG:
                idx_wait(i + _LG, (u + _LG) % _NIDX)
                gather_start(i + _LG, (u + _LG) % _NIDX, (u + _LG) % _NBUF)
            gather_wait(i, u % _NIDX, u % _NBUF)
            scatter_start(i, u % _NIDX, u % _NBUF)

        for i in range(_LI):
            idx_start(i, i)
        for i in range(_LG):
            idx_wait(i, i)
            gather_start(i, i, i)

        for u in range(_GRP):
            emit_chunk(u, u, True, False)

        def group(g, carry):
            b = g * _GRP
            for u in range(_GRP):
                emit_chunk(b + u, u, False, False)
            return carry

        lax.fori_loop(1, nchunk // _GRP - 1, group, 0)

        b = nchunk - _GRP
        for u in range(_GRP):
            emit_chunk(b + u, u, False, True)
        for i in range(nchunk - _LS, nchunk):
            scatter_wait(i, i % _NIDX, i % _NBUF)

    @pl.when(c == 0)
    def _():
        pltpu.sync_copy(init_hbm.at[pl.ds(s * _RPT, _RPT)],
                        acc.at[pl.ds(s * _RPT, _RPT)])
        plsc.subcore_barrier()

        pipeline(_NCH, s * _NCH)

        plsc.subcore_barrier()
        pltpu.sync_copy(acc.at[pl.ds(s * _RPT, _RPT)],
                        out_hbm.at[pl.ds(s * _RPT, _RPT)])


_SC_SEGSUM_CACHE = []


def _sc_segsum(x, src_p, dst_p, init):
    if not _SC_SEGSUM_CACHE:
        _SC_SEGSUM_CACHE.append(pl.kernel(
            _sc_segsum_body,
            out_type=jax.ShapeDtypeStruct((_NPAD, _D), jnp.float32),
            mesh=plsc.VectorSubcoreMesh(core_axis_name="c",
                                        subcore_axis_name="s"),
            scratch_types=[
                pltpu.VMEM((_NIDX, _CHUNK), jnp.int32),
                pltpu.VMEM((_NIDX, _CHUNK), jnp.int32),
                pltpu.VMEM((_NBUF, _CHUNK, _D), jnp.float32),
                pltpu.VMEM_SHARED((_NPAD, _D), jnp.float32),
                [pltpu.SemaphoreType.DMA] * _NBUF,
                [pltpu.SemaphoreType.DMA] * _NBUF,
                [pltpu.SemaphoreType.DMA] * _NIDX,
            ],
        ))
    return _SC_SEGSUM_CACHE[0](x, src_p, dst_p, init)


def _row_mask():
    return (lax.broadcasted_iota(jnp.int32, (_NPAD, 1), 0) < _N).astype(jnp.float32)


def _tc_root_body(x_ref, pa_ref, Wo, br, r_ref, pac_ref):
    r_ref[...] = (jnp.dot(x_ref[...], Wo[...],
                          preferred_element_type=jnp.float32) + br[...])
    pac_ref[...] = pa_ref[...]


def _conv_norm(agg_ref, r_ref, Wr, gnw, gnb, gna):
    t = jnp.dot(agg_ref[...], Wr[...],
                preferred_element_type=jnp.float32) + r_ref[...]
    mask = _row_mask()
    mean = jnp.sum(t * mask, axis=0, keepdims=True) * (1.0 / _N)
    xc = t - gna[...] * mean
    xcm = xc * mask
    var = jnp.sum(xcm * xcm, axis=0, keepdims=True) * (1.0 / _N)
    return gnw[...] * xc * lax.rsqrt(var + 1e-5) + gnb[...]


def _tc_mid_body(agg_ref, r_ref, Wr, gnw, gnb, gna, o_ref):
    y = _conv_norm(agg_ref, r_ref, Wr, gnw, gnb, gna)
    o_ref[...] = jnp.where(y >= 0, y, 0.1 * y)


def _tc_fin_body(agg_ref, r_ref, feat_ref, Wr, gnw, gnb, gna,
                 Wlt, bl, o_ref):
    y = _conv_norm(agg_ref, r_ref, Wr, gnw, gnb, gna)
    z = jnp.maximum(feat_ref[...] + y, 0.0)
    pooled = jnp.sum(z * _row_mask(), axis=0, keepdims=True) * (1.0 / _N)
    out = jnp.dot(pooled, Wlt[...], preferred_element_type=jnp.float32) + bl[...]
    o_ref[...] = jnp.maximum(out, 0.0)


_tc_root = pl.pallas_call(
    _tc_root_body,
    out_shape=(jax.ShapeDtypeStruct((_NPAD, _D), jnp.float32),
               jax.ShapeDtypeStruct((_NPAD, _D), jnp.float32)),
)

_tc_mid = pl.pallas_call(
    _tc_mid_body,
    out_shape=jax.ShapeDtypeStruct((_NPAD, _D), jnp.float32),
)

_tc_fin = pl.pallas_call(
    _tc_fin_body,
    out_shape=jax.ShapeDtypeStruct((1, _D), jnp.float32),
)


def kernel(edge_index, feat,
           W_rel0, b_rel0, W_root0, gn_w0, gn_b0, gn_a0,
           W_rel1, b_rel1, W_root1, gn_w1, gn_b1, gn_a1,
           W_rel2, b_rel2, W_root2, gn_w2, gn_b2, gn_a2,
           W_rel3, b_rel3, W_root3, gn_w3, gn_b3, gn_a3,
           W_lin, b_lin):
    src = edge_index[0].astype(jnp.int32)
    dst = edge_index[1].astype(jnp.int32)
    pad = _EPAD - _E
    src_p = jnp.concatenate(
        [src, jnp.zeros((pad,), jnp.int32)]).reshape(_NCALLS, _EPAD // _CHUNK // _NCALLS, _CHUNK)
    dst_p = jnp.concatenate(
        [dst, jnp.full((pad,), _N, jnp.int32)]).reshape(_NCALLS, _EPAD // _CHUNK // _NCALLS, _CHUNK)
    feat_p = jnp.concatenate(
        [feat, jnp.zeros((_NPAD - _N, _D), jnp.float32)], axis=0)

    convs = [(W_rel0, b_rel0, W_root0), (W_rel1, b_rel1, W_root1),
             (W_rel2, b_rel2, W_root2), (W_rel3, b_rel3, W_root3)]
    norms = [(gn_w0, gn_b0, gn_a0), (gn_w1, gn_b1, gn_a1),
             (gn_w2, gn_b2, gn_a2), (gn_w3, gn_b3, gn_a3)]

    zero = jnp.zeros((_NPAD, _D), jnp.float32)

    def segsum(x, Wo, br):
        pa = _sc_segsum(x, src_p[0], dst_p[0], zero)
        r, pac = _tc_root(x, pa, Wo, br.reshape(1, _D))
        agg = _sc_segsum(x, src_p[1], dst_p[1], pac)
        return agg, r

    x = feat_p
    for i in range(3):
        Wr, br, Wo = convs[i]
        w, b, a = norms[i]
        agg, r = segsum(x, Wo, br)
        x = _tc_mid(agg, r, Wr,
                    w.reshape(1, _D), b.reshape(1, _D), a.reshape(1, _D))

    Wr, br, Wo = convs[3]
    w, b, a = norms[3]
    agg, r = segsum(x, Wo, br)
    Wlt = jnp.zeros((_D, _D), jnp.float32).at[:, :_OUT].set(W_lin.T)
    blp = jnp.zeros((1, _D), jnp.float32).at[0, :_OUT].set(b_lin)
    out = _tc_fin(agg, r, feat_p, Wr,
                  w.reshape(1, _D), b.reshape(1, _D), a.reshape(1, _D),
                  Wlt, blp)
    return out[0, :_OUT]


# speedup vs baseline: 1.1652x; 1.1652x over previous
import jax
import jax.numpy as jnp
from jax import lax
from jax.experimental import pallas as pl
from jax.experimental.pallas import tpu as pltpu
from jax.experimental.pallas import tpu_sc as plsc

_N = 10000
_D = 128
_E = 320000
_OUT = 40

_NC = 2
_NS = 16
_NW = _NC * _NS
_CHUNK = 128
_EPAD = ((_E + _NW * _CHUNK - 1) // (_NW * _CHUNK)) * (_NW * _CHUNK)
_NCHUNK = _EPAD // (_NW * _CHUNK)
_EPT = _NCHUNK * _CHUNK
_NPAD = ((_N + _NS * 8 - 1) // (_NS * 8)) * (_NS * 8)
_RPT = _NPAD // _NS


def _sc_segsum_body(x_hbm, src_hbm, dst_hbm, zero_hbm, out_hbm,
                    sidx, didx, rows, acc, sem):
    c = lax.axis_index("c")
    s = lax.axis_index("s")
    wid = c * _NS + s
    pltpu.sync_copy(zero_hbm.at[pl.ds(s * _RPT, _RPT)],
                    acc.at[pl.ds(s * _RPT, _RPT)])
    plsc.subcore_barrier()
    base = wid * _EPT

    def body(i, carry):
        off = base + i * _CHUNK
        pltpu.sync_copy(src_hbm.at[pl.ds(off, _CHUNK)], sidx)
        pltpu.sync_copy(dst_hbm.at[pl.ds(off, _CHUNK)], didx)
        pltpu.async_copy(x_hbm.at[sidx], rows, sem).wait()
        pltpu.sync_copy(rows, acc.at[didx], add=True)
        return carry

    lax.fori_loop(0, _NCHUNK, body, 0)
    plsc.subcore_barrier()
    pltpu.sync_copy(acc.at[pl.ds(s * _RPT, _RPT)],
                    out_hbm.at[c].at[pl.ds(s * _RPT, _RPT)])


_SC_SEGSUM_CACHE = []


def _sc_segsum(x, src_p, dst_p, zero):
    if not _SC_SEGSUM_CACHE:
        _SC_SEGSUM_CACHE.append(pl.kernel(
            _sc_segsum_body,
            out_type=jax.ShapeDtypeStruct((_NC, _NPAD, _D), jnp.float32),
            mesh=plsc.VectorSubcoreMesh(core_axis_name="c",
                                        subcore_axis_name="s"),
            scratch_types=[
                pltpu.VMEM((_CHUNK,), jnp.int32),
                pltpu.VMEM((_CHUNK,), jnp.int32),
                pltpu.VMEM((_CHUNK, _D), jnp.float32),
                pltpu.VMEM_SHARED((_NPAD, _D), jnp.float32),
                pltpu.SemaphoreType.DMA,
            ],
        ))
    return _SC_SEGSUM_CACHE[0](x, src_p, dst_p, zero)


def _row_mask():
    return (lax.broadcasted_iota(jnp.int32, (_NPAD, 1), 0) < _N).astype(jnp.float32)


def _conv_norm(agg2, x, Wr, br, Wo, gnw, gnb, gna):
    agg = agg2[0] + agg2[1]
    t = (jnp.dot(agg, Wr[...], preferred_element_type=jnp.float32) + br[...]
         + jnp.dot(x[...], Wo[...], preferred_element_type=jnp.float32))
    mask = _row_mask()
    mean = jnp.sum(t * mask, axis=0, keepdims=True) * (1.0 / _N)
    xc = t - gna[...] * mean
    xcm = xc * mask
    var = jnp.sum(xcm * xcm, axis=0, keepdims=True) * (1.0 / _N)
    return gnw[...] * xc * lax.rsqrt(var + 1e-5) + gnb[...]


def _tc_mid_body(agg2_ref, x_ref, Wr, br, Wo, gnw, gnb, gna, o_ref):
    y = _conv_norm(agg2_ref[...], x_ref, Wr, br, Wo, gnw, gnb, gna)
    o_ref[...] = jnp.where(y >= 0, y, 0.1 * y)


def _tc_fin_body(agg2_ref, x_ref, feat_ref, Wr, br, Wo, gnw, gnb, gna,
                 Wlt, bl, o_ref):
    y = _conv_norm(agg2_ref[...], x_ref, Wr, br, Wo, gnw, gnb, gna)
    z = jnp.maximum(feat_ref[...] + y, 0.0)
    pooled = jnp.sum(z * _row_mask(), axis=0, keepdims=True) * (1.0 / _N)
    out = jnp.dot(pooled, Wlt[...], preferred_element_type=jnp.float32) + bl[...]
    o_ref[...] = jnp.maximum(out, 0.0)


_tc_mid = pl.pallas_call(
    _tc_mid_body,
    out_shape=jax.ShapeDtypeStruct((_NPAD, _D), jnp.float32),
)

_tc_fin = pl.pallas_call(
    _tc_fin_body,
    out_shape=jax.ShapeDtypeStruct((1, _D), jnp.float32),
)


def kernel(edge_index, feat,
           W_rel0, b_rel0, W_root0, gn_w0, gn_b0, gn_a0,
           W_rel1, b_rel1, W_root1, gn_w1, gn_b1, gn_a1,
           W_rel2, b_rel2, W_root2, gn_w2, gn_b2, gn_a2,
           W_rel3, b_rel3, W_root3, gn_w3, gn_b3, gn_a3,
           W_lin, b_lin):
    src = edge_index[0].astype(jnp.int32)
    dst = edge_index[1].astype(jnp.int32)
    pad = _EPAD - _E
    src_p = jnp.concatenate([src, jnp.zeros((pad,), jnp.int32)])
    dst_p = jnp.concatenate([dst, jnp.full((pad,), _N, jnp.int32)])
    feat_p = jnp.concatenate(
        [feat, jnp.zeros((_NPAD - _N, _D), jnp.float32)], axis=0)
    zero = jnp.zeros((_NPAD, _D), jnp.float32)

    convs = [(W_rel0, b_rel0, W_root0), (W_rel1, b_rel1, W_root1),
             (W_rel2, b_rel2, W_root2), (W_rel3, b_rel3, W_root3)]
    norms = [(gn_w0, gn_b0, gn_a0), (gn_w1, gn_b1, gn_a1),
             (gn_w2, gn_b2, gn_a2), (gn_w3, gn_b3, gn_a3)]

    x = feat_p
    for i in range(3):
        Wr, br, Wo = convs[i]
        w, b, a = norms[i]
        part = _sc_segsum(x, src_p, dst_p, zero)
        x = _tc_mid(part, x, Wr, br.reshape(1, _D), Wo,
                    w.reshape(1, _D), b.reshape(1, _D), a.reshape(1, _D))

    Wr, br, Wo = convs[3]
    w, b, a = norms[3]
    part = _sc_segsum(x, src_p, dst_p, zero)
    Wlt = jnp.zeros((_D, _D), jnp.float32).at[:, :_OUT].set(W_lin.T)
    blp = jnp.zeros((1, _D), jnp.float32).at[0, :_OUT].set(b_lin)
    out = _tc_fin(part, x, feat_p, Wr, br.reshape(1, _D), Wo,
                  w.reshape(1, _D), b.reshape(1, _D), a.reshape(1, _D),
                  Wlt, blp)
    return out[0, :_OUT]
